# baseline (speedup 1.0000x reference)
import functools

import jax
import jax.numpy as jnp
from jax import lax
from jax.experimental import pallas as pl
from jax.experimental.pallas import tpu as pltpu
from jax.experimental.pallas import tpu_sc as plsc

CD, CH, CW = 16, 32, 32
N_VOX = CD * CH * CW
N_KEYS = 4096
LATENT = 8
TOPK = 10
ALPHA = 10.0
NWORKERS = 32
QPW = N_VOX // NWORKERS

INF = float("inf")


def _coords():
    n = lax.broadcasted_iota(jnp.int32, (1, N_VOX), 1)
    return n % CW, (n // CW) % CH, n // (CW * CH)


def _shift(x, s):
    c, n = x.shape
    if s == 0:
        return x
    z = jnp.zeros((c, abs(s)), x.dtype)
    if s > 0:
        return jnp.concatenate([x[:, s:], z], axis=1)
    return jnp.concatenate([z, x[:, : n + s]], axis=1)


def _conv3x3(x, wf, cout, xc, yc, zc):
    acc = None
    t = 0
    for dz in (-1, 0, 1):
        mz = (zc + dz >= 0) & (zc + dz <= CD - 1)
        for dy in (-1, 0, 1):
            my = (yc + dy >= 0) & (yc + dy <= CH - 1)
            for dx in (-1, 0, 1):
                mx = (xc + dx >= 0) & (xc + dx <= CW - 1)
                m = mz & my & mx
                s = dz * (CH * CW) + dy * CW + dx
                xs = jnp.where(m, _shift(x, s), 0.0)
                w = wf[t * cout : (t + 1) * cout, :]
                p = jnp.dot(w, xs, preferred_element_type=jnp.float32)
                acc = p if acc is None else acc + p
                t += 1
    return acc


def _bn_relu(h, g, b):
    m = jnp.mean(h, axis=1, keepdims=True)
    v = jnp.mean((h - m) * (h - m), axis=1, keepdims=True)
    return jnp.maximum((h - m) * lax.rsqrt(v + 1e-5) * g + b, 0.0)


def _enc_body(x_ref, w1_ref, b1_ref, g1_ref, be1_ref, w2_ref, b2_ref, g2_ref,
              be2_ref, w3_ref, b3_ref, kt_ref, qa_ref, ka_ref):
    xc, yc, zc = _coords()
    h = _conv3x3(x_ref[...], w1_ref[...], 16, xc, yc, zc) + b1_ref[...]
    h = _bn_relu(h, g1_ref[...], be1_ref[...])
    h = _conv3x3(h, w2_ref[...], 32, xc, yc, zc) + b2_ref[...]
    h = _bn_relu(h, g2_ref[...], be2_ref[...])
    lat = jnp.dot(w3_ref[...], h, preferred_element_type=jnp.float32) + b3_ref[...]
    nrm = jnp.sqrt(jnp.sum(lat * lat, axis=0, keepdims=True))
    lat = lat / jnp.maximum(nrm, 1e-12)
    qn = jnp.sum(lat * lat, axis=0, keepdims=True)
    qa = jnp.concatenate([-2.0 * lat, qn, jnp.zeros((7, N_VOX), jnp.float32)],
                         axis=0)
    qa_ref[...] = qa.T
    kt = kt_ref[...]
    kn = jnp.sum(kt * kt, axis=0, keepdims=True)
    ka_ref[...] = jnp.concatenate([kt, kn], axis=0)


def _sc_retrieve(qa_flat, ka_flat, labels):
    mesh = plsc.VectorSubcoreMesh(core_axis_name="c", subcore_axis_name="s")

    @functools.partial(
        pl.kernel,
        mesh=mesh,
        compiler_params=pltpu.CompilerParams(needs_layout_passes=False),
        out_type=jax.ShapeDtypeStruct((N_VOX,), jnp.float32),
        scratch_types=[
            pltpu.VMEM((9 * N_KEYS,), jnp.float32),
            pltpu.VMEM((N_KEYS,), jnp.float32),
            pltpu.VMEM((QPW * 16,), jnp.float32),
            pltpu.VMEM((QPW,), jnp.float32),
        ],
    )
    def run(qa_hbm, ka_hbm, lab_hbm, out_hbm, ka_v, lab_v, q_v, out_v):
        wid = lax.axis_index("s") * 2 + lax.axis_index("c")
        base = wid * QPW
        pltpu.sync_copy(ka_hbm, ka_v)
        pltpu.sync_copy(lab_hbm, lab_v)
        pltpu.sync_copy(qa_hbm.at[pl.ds(base * 16, QPW * 16)], q_v)

        lane = lax.iota(jnp.int32, 16)
        lt10 = lane < TOPK

        def extract(vec, i):
            return jnp.max(jnp.where(lane == i, vec, -INF))

        def insert_chunk(d, labc, top, labv, tmax):
            def wcond(wc):
                m, t, lv, tm = wc
                return jnp.any(m & (d < tm))

            def wbody(wc):
                m, t, lv, tm = wc
                elig = m & (d < tm)
                j0 = plsc.all_reduce_ffs(elig)
                sel = lane == j0
                vv = jnp.max(jnp.where(sel, d, -INF))
                vl = jnp.max(jnp.where(sel, labc, -INF))
                mt = (t == tm) & lt10
                r0 = plsc.all_reduce_ffs(mt)
                sel2 = lane == r0
                t2 = jnp.where(sel2, vv, t)
                lv2 = jnp.where(sel2, vl, lv)
                tm2 = jnp.max(jnp.where(lt10, t2, -INF))
                return (m & jnp.logical_not(sel), t2, lv2, tm2)

            m0 = jnp.full((16,), True)
            _, t2, lv2, tm2 = lax.while_loop(wcond, wbody, (m0, top, labv, tmax))
            return (t2, lv2, tm2)

        def pair(qofs_a, qofs_b):
            rowa = q_v[pl.ds(qofs_a * 16, 16)]
            rowb = q_v[pl.ds(qofs_b * 16, 16)]
            qs_a = [extract(rowa, c) for c in range(LATENT)]
            qs_b = [extract(rowb, c) for c in range(LATENT)]
            qn_a = extract(rowa, LATENT)
            qn_b = extract(rowb, LATENT)

            def dist_tree(qs, kc, kn):
                p = [qs[c] * kc[c] for c in range(LATENT)]
                s01 = (p[0] + p[1]) + (p[2] + p[3])
                s23 = (p[4] + p[5]) + (p[6] + p[7])
                return kn + (s01 + s23)

            def gbody(g, cc):
                ta, la, ma, tb, lb, mb = cc
                gb = g * 64
                da = []
                db = []
                for k in range(4):
                    cb = gb + k * 16
                    kn = ka_v[pl.ds(LATENT * N_KEYS + cb, 16)]
                    kc = [ka_v[pl.ds(c * N_KEYS + cb, 16)] for c in range(LATENT)]
                    da.append(dist_tree(qs_a, kc, kn))
                    db.append(dist_tree(qs_b, kc, kn))
                dmina = jnp.minimum(jnp.minimum(da[0], da[1]),
                                    jnp.minimum(da[2], da[3]))
                dminb = jnp.minimum(jnp.minimum(db[0], db[1]),
                                    jnp.minimum(db[2], db[3]))

                def slow(ds_, t, l, m):
                    for k in range(4):
                        labc = lab_v[pl.ds(gb + k * 16, 16)]
                        t, l, m = lax.cond(
                            jnp.any(ds_[k] < m), insert_chunk,
                            lambda d_, l_, a, b, c: (a, b, c),
                            ds_[k], labc, t, l, m)
                    return (t, l, m)

                ta, la, ma = lax.cond(jnp.any(dmina < ma),
                                      lambda t, l, m: slow(da, t, l, m),
                                      lambda t, l, m: (t, l, m), ta, la, ma)
                tb, lb, mb = lax.cond(jnp.any(dminb < mb),
                                      lambda t, l, m: slow(db, t, l, m),
                                      lambda t, l, m: (t, l, m), tb, lb, mb)
                return (ta, la, ma, tb, lb, mb)

            top0 = jnp.full((16,), INF, jnp.float32)
            lab0 = jnp.zeros((16,), jnp.float32)
            ta, la, _, tb, lb, _ = lax.fori_loop(
                0, N_KEYS // 64, gbody,
                (top0, lab0, jnp.float32(INF), top0, lab0, jnp.float32(INF)))

            def combine(top, labv, qn):
                w = jnp.where(lt10, jnp.exp(-ALPHA * (top + qn)), 0.0)
                wsum = jnp.sum(w)
                wl = jnp.sum(w * labv)
                ones = jnp.full((16,), 1.0, jnp.float32)
                return (wl * ones) / (wsum * ones + 1e-8)

            return combine(ta, la, qn_a), combine(tb, lb, qn_b)

        def tbody(t, carry):
            res = jnp.zeros((16,), jnp.float32)
            for pp in range(8):
                pa, pb = pair(t * 16 + 2 * pp, t * 16 + 2 * pp + 1)
                res = jnp.where(lane == 2 * pp, pa, res)
                res = jnp.where(lane == 2 * pp + 1, pb, res)
            out_v[pl.ds(t * 16, 16)] = res
            return carry

        lax.fori_loop(0, QPW // 16, tbody, 0)
        pltpu.sync_copy(out_v, out_hbm.at[pl.ds(base, QPW)])

    return run(qa_flat, ka_flat, labels)


def kernel(bg_prob, ed_prob, w1, b1, g1, be1, w2, b2, g2, be2, w3, b3,
           key_store, store_labels, context_mask, add_mode):
    x = jnp.concatenate([bg_prob, ed_prob], axis=1).reshape(2, N_VOX)
    w1f = jnp.transpose(w1.reshape(16, 2, 27), (2, 0, 1)).reshape(27 * 16, 2)
    w2f = jnp.transpose(w2.reshape(32, 16, 27), (2, 0, 1)).reshape(27 * 32, 16)
    w3f = w3.reshape(LATENT, 32)
    b1c, g1c, be1c = b1.reshape(16, 1), g1.reshape(16, 1), be1.reshape(16, 1)
    b2c, g2c, be2c = b2.reshape(32, 1), g2.reshape(32, 1), be2.reshape(32, 1)
    b3c = b3.reshape(LATENT, 1)
    kt = key_store.T

    qa, ka = pl.pallas_call(
        _enc_body,
        out_shape=(
            jax.ShapeDtypeStruct((N_VOX, 16), jnp.float32),
            jax.ShapeDtypeStruct((LATENT + 1, N_KEYS), jnp.float32),
        ),
    )(x, w1f, b1c, g1c, be1c, w2f, b2c, g2c, be2c, w3f, b3c, kt)

    prob = _sc_retrieve(qa.reshape(-1), ka.reshape(-1), store_labels)
    return prob.reshape(1, CD, CH, CW)
